# Initial kernel scaffold; baseline (speedup 1.0000x reference)
import jax
import jax.numpy as jnp
from jax.experimental import pallas as pl


def kernel(input):
    raise NotImplementedError("write your pallas kernel here")


# speedup vs baseline: 218.9609x; 218.9609x over previous
import functools

import jax
import jax.numpy as jnp
from jax import lax
from jax.experimental import pallas as pl
from jax.experimental.pallas import tpu as pltpu
from jax.experimental.pallas import tpu_sc as plsc

NUM_BINS = 256
PAD = 272
LANES = 16
NC, NS = 2, 16
NW = NC * NS


def _make_hist_call(rows, pix):
    assert rows % NW == 0 and pix % LANES == 0
    rpw = rows // NW
    vecs = pix // LANES
    assert rpw % 2 == 0

    mesh = plsc.VectorSubcoreMesh(core_axis_name="c", subcore_axis_name="s")

    @functools.partial(
        pl.kernel,
        out_type=jax.ShapeDtypeStruct((rows, NUM_BINS), jnp.float32),
        mesh=mesh,
        compiler_params=pltpu.CompilerParams(needs_layout_passes=False),
        scratch_types=[
            pltpu.VMEM((pix,), jnp.float32),
            pltpu.VMEM((pix,), jnp.float32),
            pltpu.VMEM((LANES * PAD,), jnp.float32),
            pltpu.VMEM((LANES * PAD,), jnp.float32),
            pltpu.VMEM((PAD,), jnp.float32),
            pltpu.VMEM((PAD,), jnp.float32),
            pltpu.VMEM((NUM_BINS,), jnp.float32),
            pltpu.SemaphoreType.DMA,
            pltpu.SemaphoreType.DMA,
        ],
    )
    def hist_call(x_hbm, out_hbm, buf0, buf1, cnt, ssum, rcnt, rs, orow,
                  sem0, sem1):
        wid = lax.axis_index("s") * NC + lax.axis_index("c")
        row0 = wid * rpw
        lane16 = lax.iota(jnp.int32, LANES)
        lane_off = lane16 * PAD + 2
        ones = jnp.ones((LANES,), jnp.float32)
        zeros = jnp.zeros((LANES,), jnp.float32)

        pltpu.async_copy(x_hbm.at[row0], buf0, sem0)
        pltpu.async_copy(x_hbm.at[row0 + 1], buf1, sem1)

        def do_row(r, buf, sem):
            pltpu.make_async_copy(x_hbm.at[r], buf, sem).wait()

            def zbody(i, c):
                cnt[pl.ds(i * LANES, LANES)] = zeros
                ssum[pl.ds(i * LANES, LANES)] = zeros
                return c
            lax.fori_loop(0, (LANES * PAD) // LANES, zbody, 0)

            def sbody(i, c):
                v = buf[pl.ds(i * LANES, LANES)]
                iv = v.astype(jnp.int32)
                fv = v - iv.astype(jnp.float32)
                a = iv + lane_off
                plsc.addupdate_scatter(cnt, [a], ones)
                plsc.addupdate_scatter(ssum, [a], fv)
                return c
            lax.fori_loop(0, vecs, sbody, 0, unroll=8)

            @pl.when(r + 2 < row0 + rpw)
            def _():
                pltpu.async_copy(x_hbm.at[r + 2], buf, sem)

            def rbody(c, carry):
                o = c * LANES
                tc = cnt[pl.ds(o, LANES)]
                ts = ssum[pl.ds(o, LANES)]
                for l in range(1, LANES):
                    tc = tc + cnt[pl.ds(l * PAD + o, LANES)]
                    ts = ts + ssum[pl.ds(l * PAD + o, LANES)]
                rcnt[pl.ds(o, LANES)] = tc
                rs[pl.ds(o, LANES)] = ts
                return carry
            lax.fori_loop(0, PAD // LANES, rbody, 0)

            def stbody(c, carry):
                o = c * LANES
                base = lane16 + o
                c_p1 = plsc.load_gather(rcnt, [base + 3])
                c_0 = plsc.load_gather(rcnt, [base + 2])
                c_m1 = plsc.load_gather(rcnt, [base + 1])
                s_p1 = plsc.load_gather(rs, [base + 3])
                s_0 = plsc.load_gather(rs, [base + 2])
                s_m1 = plsc.load_gather(rs, [base + 1])
                s_m2 = plsc.load_gather(rs, [base])
                orow[pl.ds(o, LANES)] = (
                    c_0 + 0.5 * (c_p1 + c_m1)
                    + 0.5 * ((s_m1 + s_m2) - (s_p1 + s_0)))
                return carry
            lax.fori_loop(0, NUM_BINS // LANES, stbody, 0)

            pltpu.sync_copy(orow, out_hbm.at[r])

        def jbody(j, carry):
            r = row0 + 2 * j
            do_row(r, buf0, sem0)
            do_row(r + 1, buf1, sem1)
            return carry
        lax.fori_loop(0, rpw // 2, jbody, 0)

    return hist_call


def kernel(input):
    N, C, H, W = input.shape
    x = input.reshape(N * C, H * W)
    out = _make_hist_call(N * C, H * W)(x)
    return out.reshape(N, C, NUM_BINS)


# speedup vs baseline: 677.5245x; 3.0943x over previous
import functools

import jax
import jax.numpy as jnp
from jax import lax
from jax.experimental import pallas as pl
from jax.experimental.pallas import tpu as pltpu
from jax.experimental.pallas import tpu_sc as plsc

NUM_BINS = 256
PAD = 272
LANES = 16
NC, NS = 2, 16
NW = NC * NS


def _make_hist_call(N, C, H, W):
    rows = N * C
    assert rows % NW == 0 and W % LANES == 0
    rpw = rows // NW
    wvecs = W // LANES
    assert rpw % 2 == 0

    mesh = plsc.VectorSubcoreMesh(core_axis_name="c", subcore_axis_name="s")

    @functools.partial(
        pl.kernel,
        out_type=jax.ShapeDtypeStruct((rows, NUM_BINS), jnp.float32),
        mesh=mesh,
        compiler_params=pltpu.CompilerParams(
            needs_layout_passes=False, use_tc_tiling_on_sc=True),
        scratch_types=[
            pltpu.VMEM((H, W), jnp.float32),
            pltpu.VMEM((H, W), jnp.float32),
            pltpu.VMEM((LANES * PAD,), jnp.float32),
            pltpu.VMEM((LANES * PAD,), jnp.float32),
            pltpu.VMEM((PAD,), jnp.float32),
            pltpu.VMEM((PAD,), jnp.float32),
            pltpu.VMEM((NUM_BINS,), jnp.float32),
            pltpu.SemaphoreType.DMA,
            pltpu.SemaphoreType.DMA,
        ],
    )
    def hist_call(x_hbm, out_hbm, buf0, buf1, cnt, ssum, rcnt, rs, orow,
                  sem0, sem1):
        wid = lax.axis_index("s") * NC + lax.axis_index("c")
        row0 = wid * rpw
        lane16 = lax.iota(jnp.int32, LANES)
        lane_off = lane16 * PAD + 2
        ones = jnp.ones((LANES,), jnp.float32)
        zeros = jnp.zeros((LANES,), jnp.float32)

        bufs = (buf0, buf1)
        sems = (sem0, sem1)

        def map_copy(r, b):
            n = r // C
            c = r % C
            return pltpu.make_async_copy(x_hbm.at[n, c], bufs[b], sems[b])

        map_copy(row0, 0).start()
        map_copy(row0 + 1, 1).start()

        def do_row(r, b):
            buf = bufs[b]
            map_copy(r, b).wait()

            def zbody(i, c):
                cnt[pl.ds(i * LANES, LANES)] = zeros
                ssum[pl.ds(i * LANES, LANES)] = zeros
                return c
            lax.fori_loop(0, (LANES * PAD) // LANES, zbody, 0)

            @plsc.parallel_loop(0, H, 1, unroll=2)
            def _(h):
                for wv in range(wvecs):
                    v = buf[h, pl.ds(wv * LANES, LANES)]
                    iv = v.astype(jnp.int32)
                    fv = v - iv.astype(jnp.float32)
                    a = iv + lane_off
                    plsc.addupdate_scatter(cnt, [a], ones)
                    plsc.addupdate_scatter(ssum, [a], fv)

            @pl.when(r + 2 < row0 + rpw)
            def _():
                map_copy(r + 2, b).start()

            def rbody(c, carry):
                o = c * LANES
                tc = cnt[pl.ds(o, LANES)]
                ts = ssum[pl.ds(o, LANES)]
                for l in range(1, LANES):
                    tc = tc + cnt[pl.ds(l * PAD + o, LANES)]
                    ts = ts + ssum[pl.ds(l * PAD + o, LANES)]
                rcnt[pl.ds(o, LANES)] = tc
                rs[pl.ds(o, LANES)] = ts
                return carry
            lax.fori_loop(0, PAD // LANES, rbody, 0)

            def stbody(c, carry):
                o = c * LANES
                base = lane16 + o
                c_p1 = plsc.load_gather(rcnt, [base + 3])
                c_0 = plsc.load_gather(rcnt, [base + 2])
                c_m1 = plsc.load_gather(rcnt, [base + 1])
                s_p1 = plsc.load_gather(rs, [base + 3])
                s_0 = plsc.load_gather(rs, [base + 2])
                s_m1 = plsc.load_gather(rs, [base + 1])
                s_m2 = plsc.load_gather(rs, [base])
                orow[pl.ds(o, LANES)] = (
                    c_0 + 0.5 * (c_p1 + c_m1)
                    + 0.5 * ((s_m1 + s_m2) - (s_p1 + s_0)))
                return carry
            lax.fori_loop(0, NUM_BINS // LANES, stbody, 0)

            pltpu.sync_copy(orow, out_hbm.at[r])

        def jbody(j, carry):
            r = row0 + 2 * j
            do_row(r, 0)
            do_row(r + 1, 1)
            return carry
        lax.fori_loop(0, rpw // 2, jbody, 0)

    return hist_call


def kernel(input):
    N, C, H, W = input.shape
    out = _make_hist_call(N, C, H, W)(input)
    return out.reshape(N, C, NUM_BINS)
